# baseline (speedup 1.0000x reference)
import functools

import jax
import jax.numpy as jnp
from jax import lax
from jax.experimental import pallas as pl
from jax.experimental.pallas import tpu as pltpu
from jax.experimental.pallas import tpu_sc as plsc

_NC = 2
_NS = 16
_L = 16
_NW = _NC * _NS
_NCHUNKS = 16


def kernel(x):
    batch, seq_len, features = x.shape
    grows = batch * seq_len
    grows_pw = grows // _NW
    rows_pw = batch // _NW
    crows = grows_pw // _NCHUNKS
    cbatch = rows_pw // _NCHUNKS
    ngrp = cbatch // _L

    nelem = grows * features
    elems_pw = nelem // _NW
    celems = elems_pw // _NCHUNKS
    width = seq_len * features

    random_idxs = jax.random.randint(
        jax.random.key(1), (batch,), 0, seq_len).astype(jnp.int32)
    x2 = x.reshape(nelem)

    mesh = plsc.VectorSubcoreMesh(core_axis_name="c", subcore_axis_name="s")

    @functools.partial(
        pl.kernel,
        out_type=jax.ShapeDtypeStruct((nelem,), jnp.float32),
        mesh=mesh,
        compiler_params=pltpu.CompilerParams(needs_layout_passes=False),
        scratch_types=[
            pltpu.VMEM((celems,), jnp.float32),
            pltpu.VMEM((celems,), jnp.float32),
            pltpu.VMEM((rows_pw,), jnp.int32),
        ],
    )
    def sc_kernel(x_hbm, idx_hbm, out_hbm, bufa, bufb, idx_v):
        wid = lax.axis_index("s") * _NC + lax.axis_index("c")
        ebase = wid * elems_pw
        bbase = wid * rows_pw

        pltpu.sync_copy(idx_hbm.at[pl.ds(bbase, rows_pw)], idx_v)

        lane = lax.iota(jnp.int32, _L)
        zeros = jnp.zeros((_L,), jnp.float32)

        for c in range(_NCHUNKS):
            buf = bufa if c % 2 == 0 else bufb
            pltpu.sync_copy(x_hbm.at[pl.ds(ebase + c * celems, celems)], buf)
            for grp in range(ngrp):
                row0 = grp * _L
                rowflat = (row0 + lane) * width
                idxv = idx_v[pl.ds(c * cbatch + row0, _L)]

                def body(s8, acc, rowflat=rowflat, idxv=idxv, buf=buf):
                    for j in range(8):
                        s = s8 * 8 + j
                        v = plsc.load_gather(buf, [rowflat + (s * features + 4)])
                        acc = acc + jnp.where(idxv == s, 0.0, v)
                    return acc
                sums = lax.fori_loop(0, seq_len // 8, body,
                                     jnp.zeros((_L,), jnp.float32))
                vals = jnp.where(sums >= 2.0, jnp.float32(999.0),
                                 jnp.float32(0.0))
                grpflat = rowflat + idxv * features
                for j in range(features):
                    plsc.store_scatter(buf, [grpflat + j],
                                       vals if j == 3 else zeros)
            pltpu.sync_copy(buf, out_hbm.at[pl.ds(ebase + c * celems, celems)])

    out = sc_kernel(x2, random_idxs)
    return out.reshape(batch, seq_len, features)


# speedup vs baseline: 21.3812x; 21.3812x over previous
import functools

import jax
import jax.numpy as jnp
from jax import lax
from jax.experimental import pallas as pl
from jax.experimental.pallas import tpu as pltpu
from jax.experimental.pallas import tpu_sc as plsc

_NC = 2
_NS = 16
_L = 16
_NW = _NC * _NS
_TB = 128
_SCH = 10


def kernel(x):
    batch, seq_len, features = x.shape
    ntb = batch // _TB
    tpw = ntb // _NW
    nch = seq_len // _SCH
    lgrp = _TB // _L

    random_idxs = jax.random.randint(
        jax.random.key(1), (batch,), 0, seq_len).astype(jnp.int32)
    xv = x.reshape(ntb, _TB, seq_len, features).transpose(2, 0, 3, 1)

    mesh = plsc.VectorSubcoreMesh(core_axis_name="c", subcore_axis_name="s")

    @functools.partial(
        pl.kernel,
        out_type=jax.ShapeDtypeStruct((seq_len, ntb, features, _TB),
                                      jnp.float32),
        mesh=mesh,
        compiler_params=pltpu.CompilerParams(needs_layout_passes=False),
        scratch_types=[
            pltpu.VMEM((_SCH, tpw, features, _TB), jnp.float32),
            pltpu.VMEM((_SCH, tpw, features, _TB), jnp.float32),
            pltpu.VMEM((seq_len, _TB), jnp.float32),
            pltpu.VMEM((tpw * _TB,), jnp.int32),
            pltpu.VMEM((tpw * _TB,), jnp.float32),
        ],
    )
    def sc_kernel(x_hbm, idx_hbm, out_hbm, bufa, bufb, sbuf, idx_v, vals_v):
        wid = lax.axis_index("s") * _NC + lax.axis_index("c")
        tb0 = wid * tpw
        pltpu.sync_copy(idx_hbm.at[pl.ds(wid * tpw * _TB, tpw * _TB)], idx_v)

        lane = lax.iota(jnp.int32, _L)
        zeros = jnp.zeros((_L,), jnp.float32)

        for ti in range(tpw):
            pltpu.sync_copy(x_hbm.at[:, tb0 + ti, 4], sbuf)

            def gbody(g, _, ti=ti):
                off = ti * _TB + g * _L
                idxv = idx_v[pl.ds(off, _L)]

                def body(s, acc, idxv=idxv):
                    v = sbuf[s, pl.ds(g * _L, _L)]
                    return acc + jnp.where(idxv == s, 0.0, v)
                sums = lax.fori_loop(0, seq_len, body,
                                     jnp.zeros((_L,), jnp.float32))
                vals_v[pl.ds(off, _L)] = jnp.where(
                    sums >= 2.0, jnp.float32(999.0), jnp.float32(0.0))
                return 0
            lax.fori_loop(0, lgrp, gbody, 0)

        for c in range(nch):
            s0 = c * _SCH
            buf = bufa if c % 2 == 0 else bufb
            pltpu.sync_copy(
                x_hbm.at[pl.ds(s0, _SCH), pl.ds(tb0, tpw)], buf)

            def patch(k, _, buf=buf, s0=s0):
                off = k * _L
                idxv = idx_v[pl.ds(off, _L)]
                va = vals_v[pl.ds(off, _L)]
                mask = (idxv >= s0) & (idxv < s0 + _SCH)
                srel = idxv - s0
                tvec = jnp.zeros((_L,), jnp.int32) + k // lgrp
                lvec = (k % lgrp) * _L + lane
                for ch in range(features):
                    plsc.store_scatter(
                        buf,
                        [srel, tvec, jnp.full((_L,), ch, jnp.int32), lvec],
                        va if ch == 3 else zeros, mask=mask)
                return 0
            lax.fori_loop(0, tpw * lgrp, patch, 0)
            pltpu.sync_copy(
                buf, out_hbm.at[pl.ds(s0, _SCH), pl.ds(tb0, tpw)])

    outv = sc_kernel(xv, random_idxs)
    return outv.transpose(1, 3, 0, 2).reshape(batch, seq_len, features)


# speedup vs baseline: 28.5790x; 1.3366x over previous
import functools

import jax
import jax.numpy as jnp
from jax import lax
from jax.experimental import pallas as pl
from jax.experimental.pallas import tpu as pltpu
from jax.experimental.pallas import tpu_sc as plsc

_NC = 2
_NS = 16
_L = 16
_NW = _NC * _NS
_TB = 128
_SCH = 8
_NBUF = 3


def kernel(x):
    batch, seq_len, features = x.shape
    ntb = batch // _TB
    tpw = ntb // _NW
    nch = seq_len // _SCH
    lgrp = _TB // _L

    random_idxs = jax.random.randint(
        jax.random.key(1), (batch,), 0, seq_len).astype(jnp.int32)
    xv = x.reshape(ntb, _TB, seq_len, features).transpose(2, 0, 3, 1)

    mesh = plsc.VectorSubcoreMesh(core_axis_name="c", subcore_axis_name="s")

    @functools.partial(
        pl.kernel,
        out_type=jax.ShapeDtypeStruct((seq_len, ntb, features, _TB),
                                      jnp.float32),
        mesh=mesh,
        compiler_params=pltpu.CompilerParams(needs_layout_passes=False),
        scratch_types=[
            pltpu.VMEM((_SCH, tpw, features, _TB), jnp.float32),
            pltpu.VMEM((_SCH, tpw, features, _TB), jnp.float32),
            pltpu.VMEM((_SCH, tpw, features, _TB), jnp.float32),
            pltpu.VMEM((seq_len, _TB), jnp.float32),
            pltpu.VMEM((tpw * _TB,), jnp.int32),
            pltpu.VMEM((tpw * _TB,), jnp.float32),
            pltpu.SemaphoreType.DMA,
            pltpu.SemaphoreType.DMA,
            pltpu.SemaphoreType.DMA,
            pltpu.SemaphoreType.DMA,
            pltpu.SemaphoreType.DMA,
            pltpu.SemaphoreType.DMA,
        ],
    )
    def sc_kernel(x_hbm, idx_hbm, out_hbm, bufa, bufb, bufc, sbuf, idx_v,
                  vals_v, sia, sib, sic, soa, sob, soc):
        bufs = (bufa, bufb, bufc)
        sin = (sia, sib, sic)
        sout = (soa, sob, soc)
        wid = lax.axis_index("s") * _NC + lax.axis_index("c")
        tb0 = wid * tpw

        def src(c):
            return x_hbm.at[pl.ds(c * _SCH, _SCH), pl.ds(tb0, tpw)]

        def dst(c):
            return out_hbm.at[pl.ds(c * _SCH, _SCH), pl.ds(tb0, tpw)]

        pltpu.sync_copy(idx_hbm.at[pl.ds(wid * tpw * _TB, tpw * _TB)], idx_v)

        lane = lax.iota(jnp.int32, _L)
        zeros = jnp.zeros((_L,), jnp.float32)

        din = {0: pltpu.async_copy(src(0), bufs[0], sin[0]),
               1: pltpu.async_copy(src(1), bufs[1], sin[1])}
        dout = {}

        for ti in range(tpw):
            pltpu.sync_copy(x_hbm.at[:, tb0 + ti, 4], sbuf)

            def gbody(g, _, ti=ti):
                off = ti * _TB + g * _L
                idxv = idx_v[pl.ds(off, _L)]

                def body(s8, acc, idxv=idxv):
                    for j in range(8):
                        s = s8 * 8 + j
                        v = sbuf[s, pl.ds(g * _L, _L)]
                        acc = acc + jnp.where(idxv == s, 0.0, v)
                    return acc
                sums = lax.fori_loop(0, seq_len // 8, body,
                                     jnp.zeros((_L,), jnp.float32))
                vals_v[pl.ds(off, _L)] = jnp.where(
                    sums >= 2.0, jnp.float32(999.0), jnp.float32(0.0))
                return 0
            lax.fori_loop(0, lgrp, gbody, 0)

        for c in range(nch):
            buf = bufs[c % _NBUF]
            din[c].wait()

            def patch(k, _, buf=buf, s0=c * _SCH):
                off = k * _L
                idxv = idx_v[pl.ds(off, _L)]
                va = vals_v[pl.ds(off, _L)]
                mask = (idxv >= s0) & (idxv < s0 + _SCH)
                srel = idxv - s0
                tvec = jnp.zeros((_L,), jnp.int32) + k // lgrp
                lvec = (k % lgrp) * _L + lane
                for ch in range(features):
                    plsc.store_scatter(
                        buf,
                        [srel, tvec, jnp.full((_L,), ch, jnp.int32), lvec],
                        va if ch == 3 else zeros, mask=mask)
                return 0
            lax.fori_loop(0, tpw * lgrp, patch, 0)
            dout[c] = pltpu.async_copy(buf, dst(c), sout[c % _NBUF])
            nxt = c + 2
            if nxt < nch:
                if nxt - _NBUF >= 0:
                    dout[nxt - _NBUF].wait()
                din[nxt] = pltpu.async_copy(src(nxt), bufs[nxt % _NBUF],
                                            sin[nxt % _NBUF])
        dout[nch - 3].wait()
        dout[nch - 2].wait()
        dout[nch - 1].wait()

    outv = sc_kernel(xv, random_idxs)
    return outv.transpose(1, 3, 0, 2).reshape(batch, seq_len, features)


# speedup vs baseline: 29.3506x; 1.0270x over previous
import functools

import jax
import jax.numpy as jnp
from jax import lax
from jax.experimental import pallas as pl
from jax.experimental.pallas import tpu as pltpu
from jax.experimental.pallas import tpu_sc as plsc

_NC = 2
_NS = 16
_L = 16
_NW = _NC * _NS
_TB = 128
_SCH = 5
_NBUF = 3


def kernel(x):
    batch, seq_len, features = x.shape
    ntb = batch // _TB
    tpw = ntb // _NW
    nch = seq_len // _SCH
    lgrp = _TB // _L

    random_idxs = jax.random.randint(
        jax.random.key(1), (batch,), 0, seq_len).astype(jnp.int32)
    xv = x.reshape(ntb, _TB, seq_len, features).transpose(2, 0, 3, 1)

    mesh = plsc.VectorSubcoreMesh(core_axis_name="c", subcore_axis_name="s")

    @functools.partial(
        pl.kernel,
        out_type=jax.ShapeDtypeStruct((seq_len, ntb, features, _TB),
                                      jnp.float32),
        mesh=mesh,
        compiler_params=pltpu.CompilerParams(needs_layout_passes=False),
        scratch_types=[
            pltpu.VMEM((_SCH, tpw, features, _TB), jnp.float32),
            pltpu.VMEM((_SCH, tpw, features, _TB), jnp.float32),
            pltpu.VMEM((_SCH, tpw, features, _TB), jnp.float32),
            pltpu.VMEM((seq_len, _TB), jnp.float32),
            pltpu.VMEM((seq_len, _TB), jnp.float32),
            pltpu.VMEM((tpw * _TB,), jnp.int32),
            pltpu.VMEM((tpw * _TB,), jnp.float32),
            pltpu.SemaphoreType.DMA,
            pltpu.SemaphoreType.DMA,
            pltpu.SemaphoreType.DMA,
            pltpu.SemaphoreType.DMA,
            pltpu.SemaphoreType.DMA,
            pltpu.SemaphoreType.DMA,
            pltpu.SemaphoreType.DMA,
            pltpu.SemaphoreType.DMA,
        ],
    )
    def sc_kernel(x_hbm, idx_hbm, out_hbm, bufa, bufb, bufc, sba, sbb,
                  idx_v, vals_v, sia, sib, sic, soa, sob, soc, ssa, ssb):
        bufs = (bufa, bufb, bufc)
        sbufs = (sba, sbb)
        sin = (sia, sib, sic)
        sout = (soa, sob, soc)
        ssb_sems = (ssa, ssb)
        wid = lax.axis_index("s") * _NC + lax.axis_index("c")
        tb0 = wid * tpw

        def src(c):
            return x_hbm.at[pl.ds(c * _SCH, _SCH), pl.ds(tb0, tpw)]

        def dst(c):
            return out_hbm.at[pl.ds(c * _SCH, _SCH), pl.ds(tb0, tpw)]

        pltpu.sync_copy(idx_hbm.at[pl.ds(wid * tpw * _TB, tpw * _TB)], idx_v)

        lane = lax.iota(jnp.int32, _L)
        zeros = jnp.zeros((_L,), jnp.float32)

        din = {k: pltpu.async_copy(src(k), bufs[k], sin[k])
               for k in range(_NBUF)}
        dout = {}

        dsb = {0: pltpu.async_copy(x_hbm.at[:, tb0, 4], sba, ssa)}
        for ti in range(tpw):
            sb = sbufs[ti % 2]
            dsb[ti].wait()
            if ti + 1 < tpw:
                dsb[ti + 1] = pltpu.async_copy(
                    x_hbm.at[:, tb0 + ti + 1, 4],
                    sbufs[(ti + 1) % 2], ssb_sems[(ti + 1) % 2])

            def zbody(g, _, ti=ti, sb=sb):
                idxv = idx_v[pl.ds(ti * _TB + g * _L, _L)]
                plsc.store_scatter(sb, [idxv, g * _L + lane], zeros)
                return 0
            lax.fori_loop(0, lgrp, zbody, 0)

            def gbody(g, _, ti=ti, sb=sb):
                def body(s8, acc):
                    for j in range(8):
                        acc = acc + sb[s8 * 8 + j, pl.ds(g * _L, _L)]
                    return acc
                sums = lax.fori_loop(0, seq_len // 8, body,
                                     jnp.zeros((_L,), jnp.float32))
                vals_v[pl.ds(ti * _TB + g * _L, _L)] = jnp.where(
                    sums >= 2.0, jnp.float32(999.0), jnp.float32(0.0))
                return 0
            lax.fori_loop(0, lgrp, gbody, 0)

        for c in range(nch):
            buf = bufs[c % _NBUF]
            din[c].wait()

            def patch(k, _, buf=buf, s0=c * _SCH):
                off = k * _L
                idxv = idx_v[pl.ds(off, _L)]
                va = vals_v[pl.ds(off, _L)]
                mask = (idxv >= s0) & (idxv < s0 + _SCH)
                srel = idxv - s0
                tvec = jnp.zeros((_L,), jnp.int32) + k // lgrp
                lvec = (k % lgrp) * _L + lane
                for ch in range(features):
                    plsc.store_scatter(
                        buf,
                        [srel, tvec, jnp.full((_L,), ch, jnp.int32), lvec],
                        va if ch == 3 else zeros, mask=mask)
                return 0
            lax.fori_loop(0, tpw * lgrp, patch, 0)
            dout[c] = pltpu.async_copy(buf, dst(c), sout[c % _NBUF])
            nxt = c + 2
            if nxt < nch and nxt >= _NBUF:
                dout[nxt - _NBUF].wait()
                din[nxt] = pltpu.async_copy(src(nxt), bufs[nxt % _NBUF],
                                            sin[nxt % _NBUF])
        dout[nch - 3].wait()
        dout[nch - 2].wait()
        dout[nch - 1].wait()

    outv = sc_kernel(xv, random_idxs)
    return outv.transpose(1, 3, 0, 2).reshape(batch, seq_len, features)
